# baseline (speedup 1.0000x reference)
import jax
import jax.numpy as jnp
from jax.experimental import pallas as pl
from jax.experimental.pallas import tpu as pltpu

B = 32
CLS = 100
D = 512
CTX_LEN = 12
POOL_G = 10
POOL_A = 100
SEQ = 77
N_CTX = 36
SUF = 40
NC_SUF = 64

CLS_BLK = 20
NCB = CLS // CLS_BLK


def _prompt_kernel(idxg_ref, idxa_ref,
                   g_ref, a_ref, pref_ref, suf_ref, tok_ref,
                   out_p_ref, out_t_ref):
    b = pl.program_id(1)

    segs = []
    for k in range(3):
        i = 3 * b + k
        gi = idxg_ref[jnp.minimum(i, B - 1)]
        ai = idxa_ref[jnp.maximum(i - B, 0)]
        seg = jnp.where(i < B, g_ref[gi], a_ref[ai])
        segs.append(seg)
    ctx = jnp.concatenate(segs, axis=0)

    full = jnp.concatenate([
        pref_ref[...],
        jnp.broadcast_to(ctx[None], (CLS_BLK, N_CTX, D)),
        suf_ref[...],
    ], axis=1)
    out_p_ref[...] = full
    out_t_ref[...] = tok_ref[...]


def _nc_kernel(g_ref, ncp_ref, ncs_ref, nct_ref, out_ncp_ref, out_nct_ref):
    out_ncp_ref[...] = jnp.concatenate([
        jnp.broadcast_to(ncp_ref[...], (POOL_G, 1, D)),
        g_ref[...],
        jnp.broadcast_to(ncs_ref[...], (POOL_G, NC_SUF, D)),
    ], axis=1)
    out_nct_ref[...] = jnp.broadcast_to(nct_ref[...], (POOL_G, SEQ))


@jax.jit
def _run(idx_g, idx_a, global_prompt, attribute_prompt,
         token_prefix, token_suffix, tokenized_prompts,
         nc_token_prefix, nc_token_suffix, nc_tokenized_prompts):
    grid = (NCB, B)
    main_fn = pl.pallas_call(
        _prompt_kernel,
        grid_spec=pltpu.PrefetchScalarGridSpec(
            num_scalar_prefetch=2,
            grid=grid,
            in_specs=[
                pl.BlockSpec((POOL_G, CTX_LEN, D), lambda cb, b, *_: (0, 0, 0)),
                pl.BlockSpec((POOL_A, CTX_LEN, D), lambda cb, b, *_: (0, 0, 0)),
                pl.BlockSpec((CLS_BLK, 1, D), lambda cb, b, *_: (cb, 0, 0)),
                pl.BlockSpec((CLS_BLK, SUF, D), lambda cb, b, *_: (cb, 0, 0)),
                pl.BlockSpec((CLS_BLK, 1, SEQ), lambda cb, b, *_: (cb, 0, 0)),
            ],
            out_specs=[
                pl.BlockSpec((CLS_BLK, SEQ, D), lambda cb, b, *_: (b * NCB + cb, 0, 0)),
                pl.BlockSpec((CLS_BLK, 1, SEQ), lambda cb, b, *_: (b * NCB + cb, 0, 0)),
            ],
        ),
        out_shape=[
            jax.ShapeDtypeStruct((B * CLS, SEQ, D), jnp.float32),
            jax.ShapeDtypeStruct((B * CLS, 1, SEQ), jnp.int32),
        ],
        compiler_params=pltpu.CompilerParams(
            dimension_semantics=("parallel", "parallel")),
    )
    prompts, tok3 = main_fn(
        idx_g, idx_a, global_prompt, attribute_prompt,
        token_prefix, token_suffix,
        tokenized_prompts.reshape(CLS, 1, SEQ))

    nc_fn = pl.pallas_call(
        _nc_kernel,
        out_shape=[
            jax.ShapeDtypeStruct((POOL_G, SEQ, D), jnp.float32),
            jax.ShapeDtypeStruct((POOL_G, SEQ), jnp.int32),
        ],
    )
    nc_prompts, nc_tok = nc_fn(global_prompt, nc_token_prefix,
                               nc_token_suffix, nc_tokenized_prompts)

    return prompts, tok3.reshape(B * CLS, SEQ), nc_prompts, nc_tok


def kernel(indices_g, indices_a, global_prompt, attribute_prompt,
           token_prefix, token_suffix, tokenized_prompts,
           nc_token_prefix, nc_token_suffix, nc_tokenized_prompts):
    idx_g = indices_g.astype(jnp.int32)
    idx_a = indices_a.astype(jnp.int32)
    return _run(idx_g, idx_a, global_prompt, attribute_prompt,
                token_prefix, token_suffix, tokenized_prompts,
                nc_token_prefix, nc_token_suffix, nc_tokenized_prompts)


# speedup vs baseline: 1.0130x; 1.0130x over previous
import jax
import jax.numpy as jnp
from jax.experimental import pallas as pl
from jax.experimental.pallas import tpu as pltpu

B = 32
CLS = 100
D = 512
CTX_LEN = 12
POOL_G = 10
POOL_A = 100
SEQ = 77
N_CTX = 36
SUF = 40
NC_SUF = 64

CLS_BLK = 20
NCB = CLS // CLS_BLK


def _prompt_kernel(idxg_ref, idxa_ref,
                   g_ref, a_ref, pref_ref, suf_ref, tok_ref,
                   out_p_ref, out_t_ref):
    cb = pl.program_id(0)
    b = pl.program_id(1)
    c0 = cb * CLS_BLK

    segs = []
    for k in range(3):
        i = 3 * b + k
        gi = idxg_ref[jnp.minimum(i, B - 1)]
        ai = idxa_ref[jnp.maximum(i - B, 0)]
        seg = jnp.where(i < B, g_ref[gi], a_ref[ai])
        segs.append(seg)
    ctx = jnp.concatenate(segs, axis=0)

    full = jnp.concatenate([
        pref_ref[pl.ds(c0, CLS_BLK)],
        jnp.broadcast_to(ctx[None], (CLS_BLK, N_CTX, D)),
        suf_ref[pl.ds(c0, CLS_BLK)],
    ], axis=1)
    out_p_ref[...] = full
    out_t_ref[...] = tok_ref[pl.ds(c0, CLS_BLK)]


def _nc_kernel(g_ref, ncp_ref, ncs_ref, nct_ref, out_ncp_ref, out_nct_ref):
    out_ncp_ref[...] = jnp.concatenate([
        jnp.broadcast_to(ncp_ref[...], (POOL_G, 1, D)),
        g_ref[...],
        jnp.broadcast_to(ncs_ref[...], (POOL_G, NC_SUF, D)),
    ], axis=1)
    out_nct_ref[...] = jnp.broadcast_to(nct_ref[...], (POOL_G, SEQ))


@jax.jit
def _run(idx_g, idx_a, global_prompt, attribute_prompt,
         token_prefix, token_suffix, tokenized_prompts,
         nc_token_prefix, nc_token_suffix, nc_tokenized_prompts):
    grid = (NCB, B)
    main_fn = pl.pallas_call(
        _prompt_kernel,
        grid_spec=pltpu.PrefetchScalarGridSpec(
            num_scalar_prefetch=2,
            grid=grid,
            in_specs=[
                pl.BlockSpec((POOL_G, CTX_LEN, D), lambda cb, b, *_: (0, 0, 0)),
                pl.BlockSpec((POOL_A, CTX_LEN, D), lambda cb, b, *_: (0, 0, 0)),
                pl.BlockSpec((CLS, 1, D), lambda cb, b, *_: (0, 0, 0)),
                pl.BlockSpec((CLS, SUF, D), lambda cb, b, *_: (0, 0, 0)),
                pl.BlockSpec((CLS, 1, SEQ), lambda cb, b, *_: (0, 0, 0)),
            ],
            out_specs=[
                pl.BlockSpec((CLS_BLK, SEQ, D), lambda cb, b, *_: (b * NCB + cb, 0, 0)),
                pl.BlockSpec((CLS_BLK, 1, SEQ), lambda cb, b, *_: (b * NCB + cb, 0, 0)),
            ],
        ),
        out_shape=[
            jax.ShapeDtypeStruct((B * CLS, SEQ, D), jnp.float32),
            jax.ShapeDtypeStruct((B * CLS, 1, SEQ), jnp.int32),
        ],
        compiler_params=pltpu.CompilerParams(
            dimension_semantics=("parallel", "parallel")),
    )
    prompts, tok3 = main_fn(
        idx_g, idx_a, global_prompt, attribute_prompt,
        token_prefix, token_suffix,
        tokenized_prompts.reshape(CLS, 1, SEQ))

    nc_fn = pl.pallas_call(
        _nc_kernel,
        out_shape=[
            jax.ShapeDtypeStruct((POOL_G, SEQ, D), jnp.float32),
            jax.ShapeDtypeStruct((POOL_G, SEQ), jnp.int32),
        ],
    )
    nc_prompts, nc_tok = nc_fn(global_prompt, nc_token_prefix,
                               nc_token_suffix, nc_tokenized_prompts)

    return prompts, tok3.reshape(B * CLS, SEQ), nc_prompts, nc_tok


def kernel(indices_g, indices_a, global_prompt, attribute_prompt,
           token_prefix, token_suffix, tokenized_prompts,
           nc_token_prefix, nc_token_suffix, nc_tokenized_prompts):
    idx_g = indices_g.astype(jnp.int32)
    idx_a = indices_a.astype(jnp.int32)
    return _run(idx_g, idx_a, global_prompt, attribute_prompt,
                token_prefix, token_suffix, tokenized_prompts,
                nc_token_prefix, nc_token_suffix, nc_tokenized_prompts)


# speedup vs baseline: 1.0239x; 1.0107x over previous
import jax
import jax.numpy as jnp
from jax.experimental import pallas as pl
from jax.experimental.pallas import tpu as pltpu

B = 32
CLS = 100
D = 512
CTX_LEN = 12
POOL_G = 10
POOL_A = 100
SEQ = 77
N_CTX = 36
SUF = 40
NC_SUF = 64

CLS_BLK = 50
NCB = CLS // CLS_BLK


def _prompt_kernel(idxg_ref, idxa_ref,
                   g_ref, a_ref, pref_ref, suf_ref, tok_ref,
                   out_p_ref, out_t_ref):
    cb = pl.program_id(0)
    b = pl.program_id(1)
    c0 = cb * CLS_BLK

    segs = []
    for k in range(3):
        i = 3 * b + k
        gi = idxg_ref[jnp.minimum(i, B - 1)]
        ai = idxa_ref[jnp.maximum(i - B, 0)]
        seg = jnp.where(i < B, g_ref[gi], a_ref[ai])
        segs.append(seg)
    ctx = jnp.concatenate(segs, axis=0)

    full = jnp.concatenate([
        pref_ref[pl.ds(c0, CLS_BLK)],
        jnp.broadcast_to(ctx[None], (CLS_BLK, N_CTX, D)),
        suf_ref[pl.ds(c0, CLS_BLK)],
    ], axis=1)
    out_p_ref[...] = full
    out_t_ref[...] = tok_ref[pl.ds(c0, CLS_BLK)]


def _nc_kernel(g_ref, ncp_ref, ncs_ref, nct_ref, out_ncp_ref, out_nct_ref):
    out_ncp_ref[...] = jnp.concatenate([
        jnp.broadcast_to(ncp_ref[...], (POOL_G, 1, D)),
        g_ref[...],
        jnp.broadcast_to(ncs_ref[...], (POOL_G, NC_SUF, D)),
    ], axis=1)
    out_nct_ref[...] = jnp.broadcast_to(nct_ref[...], (POOL_G, SEQ))


@jax.jit
def _run(idx_g, idx_a, global_prompt, attribute_prompt,
         token_prefix, token_suffix, tokenized_prompts,
         nc_token_prefix, nc_token_suffix, nc_tokenized_prompts):
    grid = (NCB, B)
    main_fn = pl.pallas_call(
        _prompt_kernel,
        grid_spec=pltpu.PrefetchScalarGridSpec(
            num_scalar_prefetch=2,
            grid=grid,
            in_specs=[
                pl.BlockSpec((POOL_G, CTX_LEN, D), lambda cb, b, *_: (0, 0, 0)),
                pl.BlockSpec((POOL_A, CTX_LEN, D), lambda cb, b, *_: (0, 0, 0)),
                pl.BlockSpec((CLS, 1, D), lambda cb, b, *_: (0, 0, 0)),
                pl.BlockSpec((CLS, SUF, D), lambda cb, b, *_: (0, 0, 0)),
                pl.BlockSpec((CLS, 1, SEQ), lambda cb, b, *_: (0, 0, 0)),
            ],
            out_specs=[
                pl.BlockSpec((CLS_BLK, SEQ, D), lambda cb, b, *_: (b * NCB + cb, 0, 0)),
                pl.BlockSpec((CLS_BLK, 1, SEQ), lambda cb, b, *_: (b * NCB + cb, 0, 0)),
            ],
        ),
        out_shape=[
            jax.ShapeDtypeStruct((B * CLS, SEQ, D), jnp.float32),
            jax.ShapeDtypeStruct((B * CLS, 1, SEQ), jnp.int32),
        ],
        compiler_params=pltpu.CompilerParams(
            dimension_semantics=("parallel", "parallel")),
    )
    prompts, tok3 = main_fn(
        idx_g, idx_a, global_prompt, attribute_prompt,
        token_prefix, token_suffix,
        tokenized_prompts.reshape(CLS, 1, SEQ))

    nc_fn = pl.pallas_call(
        _nc_kernel,
        out_shape=[
            jax.ShapeDtypeStruct((POOL_G, SEQ, D), jnp.float32),
            jax.ShapeDtypeStruct((POOL_G, SEQ), jnp.int32),
        ],
    )
    nc_prompts, nc_tok = nc_fn(global_prompt, nc_token_prefix,
                               nc_token_suffix, nc_tokenized_prompts)

    return prompts, tok3.reshape(B * CLS, SEQ), nc_prompts, nc_tok


def kernel(indices_g, indices_a, global_prompt, attribute_prompt,
           token_prefix, token_suffix, tokenized_prompts,
           nc_token_prefix, nc_token_suffix, nc_tokenized_prompts):
    idx_g = indices_g.astype(jnp.int32)
    idx_a = indices_a.astype(jnp.int32)
    return _run(idx_g, idx_a, global_prompt, attribute_prompt,
                token_prefix, token_suffix, tokenized_prompts,
                nc_token_prefix, nc_token_suffix, nc_tokenized_prompts)
